# baseline (speedup 1.0000x reference)
import jax
import jax.numpy as jnp
from jax.experimental import pallas as pl
from jax.experimental.pallas import tpu as pltpu

B, D, HID, LAT, K = 16384, 512, 256, 64, 2048
BM = 256


def _dot(a, b):
    return jax.lax.dot_general(a, b, (((1,), (0,)), ((), ())),
                               preferred_element_type=jnp.float32)


def _dot_t(a, b):
    return jax.lax.dot_general(a, b, (((1,), (1,)), ((), ())),
                               preferred_element_type=jnp.float32)


def _dot_exact(a, b):
    return jax.lax.dot_general(a, b, (((1,), (0,)), ((), ())),
                               precision=jax.lax.Precision.HIGHEST,
                               preferred_element_type=jnp.float32)


def _rowsum64(a):
    s = a[:, 0:8]
    for i in range(1, 8):
        s = s + a[:, 8 * i:8 * i + 8]
    s = s[:, 0:4] + s[:, 4:8]
    s = s[:, 0:2] + s[:, 2:4]
    return s[:, 0:1] + s[:, 1:2]


def _rqvae_kernel(x_ref, w1_ref, b1_ref, w2_ref, b2_ref,
                  cb0_ref, cb1_ref, cb2_ref,
                  sq0_ref, sq1_ref, sq2_ref,
                  dw1_ref, db1_ref, dw2_ref, db2_ref,
                  c0_ref, c1_ref, c2_ref, xr_ref, q_ref,
                  r0_ref, r1_ref, r2_ref):
    x = x_ref[...]
    h = jnp.maximum(_dot(x, w1_ref[...]) + b1_ref[...], 0.0)
    z = _dot(h, w2_ref[...]) + b2_ref[...]

    iota_k = jax.lax.broadcasted_iota(jnp.int32, (BM, K), 1)
    quant = jnp.zeros_like(z)
    for cb_ref, sq_ref, c_ref, r_ref in ((cb0_ref, sq0_ref, c0_ref, r0_ref),
                                         (cb1_ref, sq1_ref, c1_ref, r1_ref),
                                         (cb2_ref, sq2_ref, c2_ref, r2_ref)):
        cb = cb_ref[...]
        resid = z - quant
        r_ref[...] = resid
        r2 = _rowsum64(resid * resid)
        c2 = sq_ref[...]
        d2 = r2 + c2 - 2.0 * _dot_t(resid, cb)
        code = jnp.argmin(d2, axis=1).astype(jnp.int32)
        c_ref[...] = code[:, None]
        oh = (iota_k == code[:, None]).astype(jnp.float32)
        quant = _dot_exact(oh, cb) + quant
        q_ref[...] = quant
        quant = q_ref[...]

    q_ref[...] = quant
    hd = jnp.maximum(_dot(quant, dw1_ref[...]) + db1_ref[...], 0.0)
    xr_ref[...] = _dot(hd, dw2_ref[...]) + db2_ref[...]


def kernel(x, enc_w1, enc_b1, enc_w2, enc_b2, cb0, cb1, cb2,
           dec_w1, dec_b1, dec_w2, dec_b2):
    grid = (B // BM,)
    row = lambda i: (i, 0)
    rep = lambda i: (0, 0)

    in_specs = [
        pl.BlockSpec((BM, D), row),
        pl.BlockSpec((D, HID), rep),
        pl.BlockSpec((1, HID), rep),
        pl.BlockSpec((HID, LAT), rep),
        pl.BlockSpec((1, LAT), rep),
        pl.BlockSpec((K, LAT), rep),
        pl.BlockSpec((K, LAT), rep),
        pl.BlockSpec((K, LAT), rep),
        pl.BlockSpec((1, K), rep),
        pl.BlockSpec((1, K), rep),
        pl.BlockSpec((1, K), rep),
        pl.BlockSpec((LAT, HID), rep),
        pl.BlockSpec((1, HID), rep),
        pl.BlockSpec((HID, D), rep),
        pl.BlockSpec((1, D), rep),
    ]
    out_specs = [
        pl.BlockSpec((BM, 1), row),
        pl.BlockSpec((BM, 1), row),
        pl.BlockSpec((BM, 1), row),
        pl.BlockSpec((BM, D), row),
        pl.BlockSpec((BM, LAT), row),
        pl.BlockSpec((BM, LAT), row),
        pl.BlockSpec((BM, LAT), row),
        pl.BlockSpec((BM, LAT), row),
    ]
    out_shape = [
        jax.ShapeDtypeStruct((B, 1), jnp.int32),
        jax.ShapeDtypeStruct((B, 1), jnp.int32),
        jax.ShapeDtypeStruct((B, 1), jnp.int32),
        jax.ShapeDtypeStruct((B, D), jnp.float32),
        jax.ShapeDtypeStruct((B, LAT), jnp.float32),
        jax.ShapeDtypeStruct((B, LAT), jnp.float32),
        jax.ShapeDtypeStruct((B, LAT), jnp.float32),
        jax.ShapeDtypeStruct((B, LAT), jnp.float32),
    ]

    c0, c1, c2, x_recon, quant, r0, r1, r2 = pl.pallas_call(
        _rqvae_kernel,
        grid=grid,
        in_specs=in_specs,
        out_specs=out_specs,
        out_shape=out_shape,
        compiler_params=pltpu.CompilerParams(
            dimension_semantics=("arbitrary",),
        ),
    )(x, enc_w1, enc_b1.reshape(1, HID), enc_w2, enc_b2.reshape(1, LAT),
      cb0, cb1, cb2,
      jnp.sum(cb0 * cb0, axis=1)[None, :],
      jnp.sum(cb1 * cb1, axis=1)[None, :],
      jnp.sum(cb2 * cb2, axis=1)[None, :],
      dec_w1, dec_b1.reshape(1, HID), dec_w2, dec_b2.reshape(1, D))

    codes = (c0.reshape(B), c1.reshape(B), c2.reshape(B))
    return (codes, x_recon, quant, (r0, r1, r2))


# speedup vs baseline: 1.4916x; 1.4916x over previous
import jax
import jax.numpy as jnp
from jax.experimental import pallas as pl
from jax.experimental.pallas import tpu as pltpu

B, D, HID, LAT, K = 16384, 512, 256, 64, 2048
BM = 256


def _dot(a, b):
    return jax.lax.dot_general(a, b, (((1,), (0,)), ((), ())),
                               preferred_element_type=jnp.float32)


def _dot_t(a, b):
    return jax.lax.dot_general(a, b, (((1,), (1,)), ((), ())),
                               preferred_element_type=jnp.float32)


def _split3(cb):
    def trunc16(a):
        return (a.view(jnp.uint32) & jnp.uint32(0xFFFF0000)).view(jnp.float32)
    hi = trunc16(cb)
    mid = trunc16(cb - hi)
    lo = cb - hi - mid
    return hi, mid, lo


def _rowsum64(a):
    s = a[:, 0:8]
    for i in range(1, 8):
        s = s + a[:, 8 * i:8 * i + 8]
    s = s[:, 0:4] + s[:, 4:8]
    s = s[:, 0:2] + s[:, 2:4]
    return s[:, 0:1] + s[:, 1:2]


def _rqvae_kernel(x_ref, w1_ref, b1_ref, w2_ref, b2_ref,
                  cb0_ref, cb1_ref, cb2_ref,
                  s0h_ref, s0m_ref, s0l_ref,
                  s1h_ref, s1m_ref, s1l_ref,
                  s2h_ref, s2m_ref, s2l_ref,
                  sq0_ref, sq1_ref, sq2_ref,
                  dw1_ref, db1_ref, dw2_ref, db2_ref,
                  c0_ref, c1_ref, c2_ref, xr_ref, q_ref,
                  r0_ref, r1_ref, r2_ref):
    x = x_ref[...]
    h = jnp.maximum(_dot(x, w1_ref[...]) + b1_ref[...], 0.0)
    z = _dot(h, w2_ref[...]) + b2_ref[...]

    iota_k = jax.lax.broadcasted_iota(jnp.int32, (BM, K), 1)
    quant = jnp.zeros_like(z)
    for cb_ref, parts, sq_ref, c_ref, r_ref in (
            (cb0_ref, (s0h_ref, s0m_ref, s0l_ref), sq0_ref, c0_ref, r0_ref),
            (cb1_ref, (s1h_ref, s1m_ref, s1l_ref), sq1_ref, c1_ref, r1_ref),
            (cb2_ref, (s2h_ref, s2m_ref, s2l_ref), sq2_ref, c2_ref, r2_ref)):
        cb = cb_ref[...]
        resid = z - quant
        r_ref[...] = resid
        r2 = _rowsum64(resid * resid)
        c2 = sq_ref[...]
        d2 = r2 + c2 - 2.0 * _dot_t(resid, cb)
        code = jnp.argmin(d2, axis=1).astype(jnp.int32)
        c_ref[...] = code[:, None]
        oh = (iota_k == code[:, None]).astype(jnp.float32)
        ph, pm, plo = parts
        quant = ((_dot(oh, ph[...]) + _dot(oh, pm[...])) +
                 _dot(oh, plo[...])) + quant
        q_ref[...] = quant
        quant = q_ref[...]

    q_ref[...] = quant
    hd = jnp.maximum(_dot(quant, dw1_ref[...]) + db1_ref[...], 0.0)
    xr_ref[...] = _dot(hd, dw2_ref[...]) + db2_ref[...]


def kernel(x, enc_w1, enc_b1, enc_w2, enc_b2, cb0, cb1, cb2,
           dec_w1, dec_b1, dec_w2, dec_b2):
    grid = (B // BM,)
    row = lambda i: (i, 0)
    rep = lambda i: (0, 0)

    in_specs = [
        pl.BlockSpec((BM, D), row),
        pl.BlockSpec((D, HID), rep),
        pl.BlockSpec((1, HID), rep),
        pl.BlockSpec((HID, LAT), rep),
        pl.BlockSpec((1, LAT), rep),
        pl.BlockSpec((K, LAT), rep),
        pl.BlockSpec((K, LAT), rep),
        pl.BlockSpec((K, LAT), rep),
        pl.BlockSpec((K, LAT), rep),
        pl.BlockSpec((K, LAT), rep),
        pl.BlockSpec((K, LAT), rep),
        pl.BlockSpec((K, LAT), rep),
        pl.BlockSpec((K, LAT), rep),
        pl.BlockSpec((K, LAT), rep),
        pl.BlockSpec((K, LAT), rep),
        pl.BlockSpec((K, LAT), rep),
        pl.BlockSpec((K, LAT), rep),
        pl.BlockSpec((1, K), rep),
        pl.BlockSpec((1, K), rep),
        pl.BlockSpec((1, K), rep),
        pl.BlockSpec((LAT, HID), rep),
        pl.BlockSpec((1, HID), rep),
        pl.BlockSpec((HID, D), rep),
        pl.BlockSpec((1, D), rep),
    ]
    out_specs = [
        pl.BlockSpec((BM, 1), row),
        pl.BlockSpec((BM, 1), row),
        pl.BlockSpec((BM, 1), row),
        pl.BlockSpec((BM, D), row),
        pl.BlockSpec((BM, LAT), row),
        pl.BlockSpec((BM, LAT), row),
        pl.BlockSpec((BM, LAT), row),
        pl.BlockSpec((BM, LAT), row),
    ]
    out_shape = [
        jax.ShapeDtypeStruct((B, 1), jnp.int32),
        jax.ShapeDtypeStruct((B, 1), jnp.int32),
        jax.ShapeDtypeStruct((B, 1), jnp.int32),
        jax.ShapeDtypeStruct((B, D), jnp.float32),
        jax.ShapeDtypeStruct((B, LAT), jnp.float32),
        jax.ShapeDtypeStruct((B, LAT), jnp.float32),
        jax.ShapeDtypeStruct((B, LAT), jnp.float32),
        jax.ShapeDtypeStruct((B, LAT), jnp.float32),
    ]

    c0, c1, c2, x_recon, quant, r0, r1, r2 = pl.pallas_call(
        _rqvae_kernel,
        grid=grid,
        in_specs=in_specs,
        out_specs=out_specs,
        out_shape=out_shape,
        compiler_params=pltpu.CompilerParams(
            dimension_semantics=("arbitrary",),
        ),
    )(x, enc_w1, enc_b1.reshape(1, HID), enc_w2, enc_b2.reshape(1, LAT),
      cb0, cb1, cb2,
      *_split3(cb0), *_split3(cb1), *_split3(cb2),
      jnp.sum(cb0 * cb0, axis=1)[None, :],
      jnp.sum(cb1 * cb1, axis=1)[None, :],
      jnp.sum(cb2 * cb2, axis=1)[None, :],
      dec_w1, dec_b1.reshape(1, HID), dec_w2, dec_b2.reshape(1, D))

    codes = (c0.reshape(B), c1.reshape(B), c2.reshape(B))
    return (codes, x_recon, quant, (r0, r1, r2))


# speedup vs baseline: 1.6350x; 1.0961x over previous
import jax
import jax.numpy as jnp
from jax.experimental import pallas as pl
from jax.experimental.pallas import tpu as pltpu

B, D, HID, LAT, K = 16384, 512, 256, 64, 2048
BM = 512


def _dot(a, b):
    return jax.lax.dot_general(a, b, (((1,), (0,)), ((), ())),
                               preferred_element_type=jnp.float32)


def _dot_t(a, b):
    return jax.lax.dot_general(a, b, (((1,), (1,)), ((), ())),
                               preferred_element_type=jnp.float32)


def _split3(cb):
    def trunc16(a):
        return (a.view(jnp.uint32) & jnp.uint32(0xFFFF0000)).view(jnp.float32)
    hi = trunc16(cb)
    mid = trunc16(cb - hi)
    lo = cb - hi - mid
    return hi, mid, lo


def _rowsum64(a):
    s = a[:, 0:8]
    for i in range(1, 8):
        s = s + a[:, 8 * i:8 * i + 8]
    s = s[:, 0:4] + s[:, 4:8]
    s = s[:, 0:2] + s[:, 2:4]
    return s[:, 0:1] + s[:, 1:2]


def _rqvae_kernel(x_ref, w1_ref, b1_ref, w2_ref, b2_ref,
                  cb0_ref, cb1_ref, cb2_ref,
                  s0h_ref, s0m_ref, s0l_ref,
                  s1h_ref, s1m_ref, s1l_ref,
                  s2h_ref, s2m_ref, s2l_ref,
                  sq0_ref, sq1_ref, sq2_ref,
                  dw1_ref, db1_ref, dw2_ref, db2_ref,
                  c0_ref, c1_ref, c2_ref, xr_ref, q_ref,
                  r0_ref, r1_ref, r2_ref):
    x = x_ref[...]
    h = jnp.maximum(_dot(x, w1_ref[...]) + b1_ref[...], 0.0)
    z = _dot(h, w2_ref[...]) + b2_ref[...]

    iota_k = jax.lax.broadcasted_iota(jnp.int32, (BM, K), 1)
    quant = jnp.zeros_like(z)
    for cb_ref, parts, sq_ref, c_ref, r_ref in (
            (cb0_ref, (s0h_ref, s0m_ref, s0l_ref), sq0_ref, c0_ref, r0_ref),
            (cb1_ref, (s1h_ref, s1m_ref, s1l_ref), sq1_ref, c1_ref, r1_ref),
            (cb2_ref, (s2h_ref, s2m_ref, s2l_ref), sq2_ref, c2_ref, r2_ref)):
        cb = cb_ref[...]
        resid = z - quant
        r_ref[...] = resid
        r2 = _rowsum64(resid * resid)
        c2 = sq_ref[...]
        d2 = r2 + c2 - 2.0 * _dot_t(resid, cb)
        code = jnp.argmin(d2, axis=1).astype(jnp.int32)
        c_ref[...] = code[:, None]
        oh = (iota_k == code[:, None]).astype(jnp.float32)
        ph, pm, plo = parts
        quant = ((_dot(oh, ph[...]) + _dot(oh, pm[...])) +
                 _dot(oh, plo[...])) + quant
        q_ref[...] = quant
        quant = q_ref[...]

    q_ref[...] = quant
    hd = jnp.maximum(_dot(quant, dw1_ref[...]) + db1_ref[...], 0.0)
    xr_ref[...] = _dot(hd, dw2_ref[...]) + db2_ref[...]


def kernel(x, enc_w1, enc_b1, enc_w2, enc_b2, cb0, cb1, cb2,
           dec_w1, dec_b1, dec_w2, dec_b2):
    grid = (B // BM,)
    row = lambda i: (i, 0)
    rep = lambda i: (0, 0)

    in_specs = [
        pl.BlockSpec((BM, D), row),
        pl.BlockSpec((D, HID), rep),
        pl.BlockSpec((1, HID), rep),
        pl.BlockSpec((HID, LAT), rep),
        pl.BlockSpec((1, LAT), rep),
        pl.BlockSpec((K, LAT), rep),
        pl.BlockSpec((K, LAT), rep),
        pl.BlockSpec((K, LAT), rep),
        pl.BlockSpec((K, LAT), rep),
        pl.BlockSpec((K, LAT), rep),
        pl.BlockSpec((K, LAT), rep),
        pl.BlockSpec((K, LAT), rep),
        pl.BlockSpec((K, LAT), rep),
        pl.BlockSpec((K, LAT), rep),
        pl.BlockSpec((K, LAT), rep),
        pl.BlockSpec((K, LAT), rep),
        pl.BlockSpec((K, LAT), rep),
        pl.BlockSpec((1, K), rep),
        pl.BlockSpec((1, K), rep),
        pl.BlockSpec((1, K), rep),
        pl.BlockSpec((LAT, HID), rep),
        pl.BlockSpec((1, HID), rep),
        pl.BlockSpec((HID, D), rep),
        pl.BlockSpec((1, D), rep),
    ]
    out_specs = [
        pl.BlockSpec((BM, 1), row),
        pl.BlockSpec((BM, 1), row),
        pl.BlockSpec((BM, 1), row),
        pl.BlockSpec((BM, D), row),
        pl.BlockSpec((BM, LAT), row),
        pl.BlockSpec((BM, LAT), row),
        pl.BlockSpec((BM, LAT), row),
        pl.BlockSpec((BM, LAT), row),
    ]
    out_shape = [
        jax.ShapeDtypeStruct((B, 1), jnp.int32),
        jax.ShapeDtypeStruct((B, 1), jnp.int32),
        jax.ShapeDtypeStruct((B, 1), jnp.int32),
        jax.ShapeDtypeStruct((B, D), jnp.float32),
        jax.ShapeDtypeStruct((B, LAT), jnp.float32),
        jax.ShapeDtypeStruct((B, LAT), jnp.float32),
        jax.ShapeDtypeStruct((B, LAT), jnp.float32),
        jax.ShapeDtypeStruct((B, LAT), jnp.float32),
    ]

    c0, c1, c2, x_recon, quant, r0, r1, r2 = pl.pallas_call(
        _rqvae_kernel,
        grid=grid,
        in_specs=in_specs,
        out_specs=out_specs,
        out_shape=out_shape,
        compiler_params=pltpu.CompilerParams(
            dimension_semantics=("arbitrary",),
        ),
    )(x, enc_w1, enc_b1.reshape(1, HID), enc_w2, enc_b2.reshape(1, LAT),
      cb0, cb1, cb2,
      *_split3(cb0), *_split3(cb1), *_split3(cb2),
      jnp.sum(cb0 * cb0, axis=1)[None, :],
      jnp.sum(cb1 * cb1, axis=1)[None, :],
      jnp.sum(cb2 * cb2, axis=1)[None, :],
      dec_w1, dec_b1.reshape(1, HID), dec_w2, dec_b2.reshape(1, D))

    codes = (c0.reshape(B), c1.reshape(B), c2.reshape(B))
    return (codes, x_recon, quant, (r0, r1, r2))
